# baseline (speedup 1.0000x reference)
import jax
import jax.numpy as jnp
from jax import lax
from jax.experimental import pallas as pl
from jax.experimental.pallas import tpu as pltpu
from jax.experimental.pallas import tpu_sc as plsc

N_NODES = 10000
N_EDGES = 320000
N_FEAT = 128
N_CLASSES = 64

R = 10240
IDX_ROWS = N_EDGES // 128
ROWS_PER_TILE = 160
TAIL_ROWS = IDX_ROWS - 15 * ROWS_PER_TILE
BLK = 5
N_PAIR = ROWS_PER_TILE // BLK // 2
N_PAIR_TAIL = TAIL_ROWS // BLK // 2
HALF = N_CLASSES // 2
INIT_ROWS = 64
ACC_PER_TILE = R // 16


def _linear_in_body(x_ref, w_in_ref, b_in_ref, w_out_ref, z_ref):
    wf = jax.lax.dot_general(
        w_out_ref[...], w_in_ref[...],
        (((1,), (0,)), ((), ())), preferred_element_type=jnp.float32)
    b1 = jax.lax.dot_general(
        b_in_ref[...], w_out_ref[...],
        (((1,), (1,)), ((), ())), preferred_element_type=jnp.float32)
    z = jax.lax.dot_general(
        x_ref[...], wf,
        (((1,), (1,)), ((), ())), preferred_element_type=jnp.float32) + b1
    z_ref[0] = z[:, :HALF]
    z_ref[1] = z[:, HALF:]


def _linear_in(x, w_in, b_in, w_out):
    blk = 1024
    return pl.pallas_call(
        _linear_in_body,
        grid=(R // blk,),
        in_specs=[
            pl.BlockSpec((blk, N_FEAT), lambda i: (i, 0)),
            pl.BlockSpec((N_FEAT, N_FEAT), lambda i: (0, 0)),
            pl.BlockSpec((1, N_FEAT), lambda i: (0, 0)),
            pl.BlockSpec((N_CLASSES, N_FEAT), lambda i: (0, 0)),
        ],
        out_specs=pl.BlockSpec((2, blk, HALF), lambda i: (0, i, 0)),
        out_shape=jax.ShapeDtypeStruct((2, R, HALF), jnp.float32),
    )(x, w_in, b_in, w_out)


def _prop2_body(tbl, sd_hbm, bias_hbm, out_hbm,
                zb, acc1, slab, rows, init0, init1, bias_v,
                sem_i, sem_g0, sem_g1, sem_s):
    c = lax.axis_index("c")
    s = lax.axis_index("s")
    row0 = s * ROWS_PER_TILE
    acc_off = s * ACC_PER_TILE

    @pl.when(s < 15)
    def _load_full():
        pltpu.async_copy(sd_hbm.at[0, pl.ds(row0, ROWS_PER_TILE)], slab.at[0], sem_i)
        pltpu.async_copy(sd_hbm.at[1, pl.ds(row0, ROWS_PER_TILE)], slab.at[1], sem_i)

    @pl.when(s == 15)
    def _load_tail():
        pltpu.async_copy(sd_hbm.at[0, pl.ds(row0, TAIL_ROWS)],
                         slab.at[0, pl.ds(0, TAIL_ROWS)], sem_i)
        pltpu.async_copy(sd_hbm.at[1, pl.ds(row0, TAIL_ROWS)],
                         slab.at[1, pl.ds(0, TAIL_ROWS)], sem_i)

    n_pair = jnp.where(s == 15, N_PAIR_TAIL, N_PAIR)
    stage_h = pltpu.async_copy(
        tbl.at[c].at[pl.ds(acc_off, ACC_PER_TILE)],
        zb.at[pl.ds(acc_off, ACC_PER_TILE)], sem_s)

    pltpu.sync_copy(bias_hbm.at[c], bias_v)
    zero = jnp.zeros((16,), jnp.float32)
    lo = bias_v[pl.ds(0, 16)]
    hi = bias_v[pl.ds(16, 16)]
    for r in range(INIT_ROWS):
        init0[r, pl.ds(0, 16)] = zero
        init0[r, pl.ds(16, 16)] = zero
        init1[r, pl.ds(0, 16)] = lo
        init1[r, pl.ds(16, 16)] = hi
    init_hs = [
        pltpu.async_copy(
            init0, acc1.at[pl.ds(acc_off + k * INIT_ROWS, INIT_ROWS)], sem_s)
        for k in range(ACC_PER_TILE // INIT_ROWS)
    ]

    gather_sems = (sem_g0, sem_g1)

    def make_layer(src_tbl, acc):
        def fire(batch, buf, sem):
            for j in range(BLK):
                pltpu.async_copy(
                    src_tbl.at[slab.at[0, batch * BLK + j]],
                    rows.at[buf, j], sem)

        def wait_gathers(buf):
            for j in range(BLK):
                pltpu.make_async_copy(
                    src_tbl.at[pl.ds(0, 128)], rows.at[buf, j],
                    gather_sems[buf]).wait()

        def scatter(batch, buf):
            hs = [
                pltpu.async_copy(
                    rows.at[buf, j],
                    acc.at[slab.at[1, batch * BLK + j]],
                    sem_s, add=True)
                for j in range(BLK)
            ]
            for h in hs:
                h.wait()

        def loop():
            def pair(i, _):
                a = 2 * i
                fire(a + 1, 1, sem_g1)
                wait_gathers(0)
                scatter(a, 0)

                @pl.when(i < n_pair - 1)
                def _fire_next():
                    fire(a + 2, 0, sem_g0)

                wait_gathers(1)
                scatter(a + 1, 1)
                return _
            lax.fori_loop(0, n_pair, pair, None)
        return fire, loop

    fire1, loop1 = make_layer(zb, acc1)
    fire2, loop2 = make_layer(acc1, zb)

    @pl.when(s < 15)
    def _wait_full():
        for k in range(2):
            pltpu.make_async_copy(
                sd_hbm.at[0, pl.ds(0, ROWS_PER_TILE)], slab.at[k], sem_i).wait()

    @pl.when(s == 15)
    def _wait_tail():
        for k in range(2):
            pltpu.make_async_copy(
                sd_hbm.at[0, pl.ds(0, TAIL_ROWS)],
                slab.at[k, pl.ds(0, TAIL_ROWS)], sem_i).wait()

    stage_h.wait()
    for h in init_hs:
        h.wait()
    plsc.subcore_barrier()
    fire1(0, 0, sem_g0)
    loop1()
    plsc.subcore_barrier()

    fire2(0, 0, sem_g0)
    init2_hs = [
        pltpu.async_copy(
            init1, zb.at[pl.ds(acc_off + k * INIT_ROWS, INIT_ROWS)], sem_s)
        for k in range(ACC_PER_TILE // INIT_ROWS)
    ]
    for h in init2_hs:
        h.wait()
    plsc.subcore_barrier()

    loop2()
    plsc.subcore_barrier()

    fr = N_NODES // 16
    pltpu.sync_copy(
        zb.at[pl.ds(s * fr, fr)],
        out_hbm.at[pl.ds(s * fr, fr), pl.ds(c * HALF, HALF)])


def _make_prop2():
    mesh = plsc.VectorSubcoreMesh(core_axis_name="c", subcore_axis_name="s")
    return pl.kernel(
        _prop2_body,
        out_type=jax.ShapeDtypeStruct((N_NODES, N_CLASSES), jnp.float32),
        mesh=mesh,
        scratch_types=[
            pltpu.VMEM_SHARED((R, HALF), jnp.float32),
            pltpu.VMEM_SHARED((R, HALF), jnp.float32),
            pltpu.VMEM((2, ROWS_PER_TILE, 128), jnp.int32),
            pltpu.VMEM((2, BLK, 128, HALF), jnp.float32),
            pltpu.VMEM((INIT_ROWS, HALF), jnp.float32),
            pltpu.VMEM((INIT_ROWS, HALF), jnp.float32),
            pltpu.VMEM((HALF,), jnp.float32),
            pltpu.SemaphoreType.DMA,
            pltpu.SemaphoreType.DMA,
            pltpu.SemaphoreType.DMA,
            pltpu.SemaphoreType.DMA,
        ],
        compiler_params=pltpu.CompilerParams(use_tc_tiling_on_sc=False),
    )


def kernel(x, adj, W_in, b_in, W_out, b_out):
    sd = adj.reshape(2, IDX_ROWS, 128)
    bias2 = b_out.reshape(2, HALF)

    z = _linear_in(x, W_in, b_in.reshape(1, N_FEAT), W_out)
    return _make_prop2()(z, sd, bias2)


# speedup vs baseline: 1.0079x; 1.0079x over previous
import jax
import jax.numpy as jnp
from jax import lax
from jax.experimental import pallas as pl
from jax.experimental.pallas import tpu as pltpu
from jax.experimental.pallas import tpu_sc as plsc

N_NODES = 10000
N_EDGES = 320000
N_FEAT = 128
N_CLASSES = 64

R = 10240
IDX_ROWS = N_EDGES // 128
ROWS_PER_TILE = 160
TAIL_ROWS = IDX_ROWS - 15 * ROWS_PER_TILE
BLK = 5
N_PAIR = ROWS_PER_TILE // BLK // 2
N_PAIR_TAIL = TAIL_ROWS // BLK // 2
HALF = N_CLASSES // 2
INIT_ROWS = 64
ACC_PER_TILE = R // 16


def _linear_in_body(x_ref, w_in_ref, b_in_ref, w_out_ref, z_ref):
    wf = jax.lax.dot_general(
        w_out_ref[...], w_in_ref[...],
        (((1,), (0,)), ((), ())), preferred_element_type=jnp.float32)
    b1 = jax.lax.dot_general(
        b_in_ref[...], w_out_ref[...],
        (((1,), (1,)), ((), ())), preferred_element_type=jnp.float32)
    z = jax.lax.dot_general(
        x_ref[...], wf,
        (((1,), (1,)), ((), ())), preferred_element_type=jnp.float32) + b1
    z_ref[0] = z[:, :HALF]
    z_ref[1] = z[:, HALF:]


def _linear_in(x, w_in, b_in, w_out):
    blk = 1024
    return pl.pallas_call(
        _linear_in_body,
        grid=(R // blk,),
        in_specs=[
            pl.BlockSpec((blk, N_FEAT), lambda i: (i, 0)),
            pl.BlockSpec((N_FEAT, N_FEAT), lambda i: (0, 0)),
            pl.BlockSpec((1, N_FEAT), lambda i: (0, 0)),
            pl.BlockSpec((N_CLASSES, N_FEAT), lambda i: (0, 0)),
        ],
        out_specs=pl.BlockSpec((2, blk, HALF), lambda i: (0, i, 0)),
        out_shape=jax.ShapeDtypeStruct((2, R, HALF), jnp.float32),
    )(x, w_in, b_in, w_out)


def _prop2_body(tbl, sd_hbm, bias_hbm, out_hbm,
                zb, acc1, slab, rows, init0, init1, bias_v,
                sem_i, sem_g0, sem_g1, sem_s):
    c = lax.axis_index("c")
    s = lax.axis_index("s")
    row0 = s * ROWS_PER_TILE
    acc_off = s * ACC_PER_TILE

    @pl.when(s < 15)
    def _load_full():
        pltpu.async_copy(sd_hbm.at[0, pl.ds(row0, ROWS_PER_TILE)], slab.at[0], sem_i)
        pltpu.async_copy(sd_hbm.at[1, pl.ds(row0, ROWS_PER_TILE)], slab.at[1], sem_i)

    @pl.when(s == 15)
    def _load_tail():
        pltpu.async_copy(sd_hbm.at[0, pl.ds(row0, TAIL_ROWS)],
                         slab.at[0, pl.ds(0, TAIL_ROWS)], sem_i)
        pltpu.async_copy(sd_hbm.at[1, pl.ds(row0, TAIL_ROWS)],
                         slab.at[1, pl.ds(0, TAIL_ROWS)], sem_i)

    n_pair = jnp.where(s == 15, N_PAIR_TAIL, N_PAIR)
    stage_h = pltpu.async_copy(
        tbl.at[c].at[pl.ds(acc_off, ACC_PER_TILE)],
        zb.at[pl.ds(acc_off, ACC_PER_TILE)], sem_s)

    pltpu.sync_copy(bias_hbm.at[c], bias_v)
    zero = jnp.zeros((16,), jnp.float32)
    lo = bias_v[pl.ds(0, 16)]
    hi = bias_v[pl.ds(16, 16)]
    for r in range(INIT_ROWS):
        init0[r, pl.ds(0, 16)] = zero
        init0[r, pl.ds(16, 16)] = zero
        init1[r, pl.ds(0, 16)] = lo
        init1[r, pl.ds(16, 16)] = hi
    init_hs = [
        pltpu.async_copy(
            init0, acc1.at[pl.ds(acc_off + k * INIT_ROWS, INIT_ROWS)], sem_s)
        for k in range(ACC_PER_TILE // INIT_ROWS)
    ]

    gather_sems = (sem_g0, sem_g1)

    def make_layer(src_tbl, acc):
        def fire(batch, buf, sem):
            for j in range(BLK):
                pltpu.async_copy(
                    src_tbl.at[slab.at[0, batch * BLK + j]],
                    rows.at[buf, j], sem)

        def wait_gathers(buf):
            for j in range(BLK):
                pltpu.make_async_copy(
                    src_tbl.at[pl.ds(0, 128)], rows.at[buf, j],
                    gather_sems[buf]).wait()

        def process(batch, buf):
            hs = []
            for j in range(BLK):
                pltpu.make_async_copy(
                    src_tbl.at[pl.ds(0, 128)], rows.at[buf, j],
                    gather_sems[buf]).wait()
                hs.append(pltpu.async_copy(
                    rows.at[buf, j],
                    acc.at[slab.at[1, batch * BLK + j]],
                    sem_s, add=True))
            for h in hs:
                h.wait()

        def loop():
            def pair(i, _):
                a = 2 * i
                fire(a + 1, 1, sem_g1)
                process(a, 0)

                @pl.when(i < n_pair - 1)
                def _fire_next():
                    fire(a + 2, 0, sem_g0)

                process(a + 1, 1)
                return _
            lax.fori_loop(0, n_pair, pair, None)
        return fire, loop

    fire1, loop1 = make_layer(zb, acc1)
    fire2, loop2 = make_layer(acc1, zb)

    @pl.when(s < 15)
    def _wait_full():
        for k in range(2):
            pltpu.make_async_copy(
                sd_hbm.at[0, pl.ds(0, ROWS_PER_TILE)], slab.at[k], sem_i).wait()

    @pl.when(s == 15)
    def _wait_tail():
        for k in range(2):
            pltpu.make_async_copy(
                sd_hbm.at[0, pl.ds(0, TAIL_ROWS)],
                slab.at[k, pl.ds(0, TAIL_ROWS)], sem_i).wait()

    stage_h.wait()
    for h in init_hs:
        h.wait()
    plsc.subcore_barrier()
    fire1(0, 0, sem_g0)
    loop1()
    plsc.subcore_barrier()

    fire2(0, 0, sem_g0)
    init2_hs = [
        pltpu.async_copy(
            init1, zb.at[pl.ds(acc_off + k * INIT_ROWS, INIT_ROWS)], sem_s)
        for k in range(ACC_PER_TILE // INIT_ROWS)
    ]
    for h in init2_hs:
        h.wait()
    plsc.subcore_barrier()

    loop2()
    plsc.subcore_barrier()

    fr = N_NODES // 16
    pltpu.sync_copy(
        zb.at[pl.ds(s * fr, fr)],
        out_hbm.at[pl.ds(s * fr, fr), pl.ds(c * HALF, HALF)])


def _make_prop2():
    mesh = plsc.VectorSubcoreMesh(core_axis_name="c", subcore_axis_name="s")
    return pl.kernel(
        _prop2_body,
        out_type=jax.ShapeDtypeStruct((N_NODES, N_CLASSES), jnp.float32),
        mesh=mesh,
        scratch_types=[
            pltpu.VMEM_SHARED((R, HALF), jnp.float32),
            pltpu.VMEM_SHARED((R, HALF), jnp.float32),
            pltpu.VMEM((2, ROWS_PER_TILE, 128), jnp.int32),
            pltpu.VMEM((2, BLK, 128, HALF), jnp.float32),
            pltpu.VMEM((INIT_ROWS, HALF), jnp.float32),
            pltpu.VMEM((INIT_ROWS, HALF), jnp.float32),
            pltpu.VMEM((HALF,), jnp.float32),
            pltpu.SemaphoreType.DMA,
            pltpu.SemaphoreType.DMA,
            pltpu.SemaphoreType.DMA,
            pltpu.SemaphoreType.DMA,
        ],
        compiler_params=pltpu.CompilerParams(use_tc_tiling_on_sc=False),
    )


def kernel(x, adj, W_in, b_in, W_out, b_out):
    sd = adj.reshape(2, IDX_ROWS, 128)
    bias2 = b_out.reshape(2, HALF)

    z = _linear_in(x, W_in, b_in.reshape(1, N_FEAT), W_out)
    return _make_prop2()(z, sd, bias2)
